# Initial kernel scaffold; baseline (speedup 1.0000x reference)
import jax
import jax.numpy as jnp
from jax.experimental import pallas as pl


def kernel(x, W):
    raise NotImplementedError("write your pallas kernel here")


# speedup vs baseline: 1.0824x; 1.0824x over previous
import functools

import numpy as np
import jax
import jax.numpy as jnp
from jax import lax
from jax.experimental import pallas as pl
from jax.experimental.pallas import tpu as pltpu
from jax.experimental.pallas import tpu_sc as plsc

VOCAB = 100000
DIM = 1024
HALF = DIM // 2
BATCH = 4
SEQ = 2048
B = BATCH * SEQ

NC, NS = 2, 16
NW = NC * NS
B_PER_W = B // NW
CH = 64
NCH = B_PER_W // CH

_sc_mesh = plsc.VectorSubcoreMesh(core_axis_name="c", subcore_axis_name="s")


@functools.partial(
    pl.kernel,
    mesh=_sc_mesh,
    out_type=jax.ShapeDtypeStruct((B, DIM), jnp.float32),
    scratch_types=[
        pltpu.VMEM((NCH, CH), jnp.int32),
        pltpu.VMEM((CH, DIM), jnp.float32),
        pltpu.SemaphoreType.DMA,
    ],
)
def _sc_gather(table_hbm, idx_hbm, out_hbm, idx_v, rows_v, sem):
    wid = lax.axis_index("s") * NC + lax.axis_index("c")
    pltpu.sync_copy(idx_hbm.at[wid], idx_v)
    for j in range(NCH):
        pltpu.async_copy(table_hbm.at[idx_v.at[j]], rows_v, sem).wait()
        pltpu.sync_copy(rows_v, out_hbm.at[pl.ds(wid * B_PER_W + j * CH, CH)])


def _rope_tables():
    fi = np.arange(HALF, dtype=np.float32)
    freqs = (1.0 / (10000.0 ** (fi / DIM))).astype(np.float32)
    pos = np.arange(SEQ, dtype=np.float32)
    angles = pos[:, None] * freqs[None, :]
    return np.sin(angles).astype(np.float32), np.cos(angles).astype(np.float32)


_SIN, _COS = _rope_tables()

RB = 256


def _rope_body(e_ref, s_ref, c_ref, o_ref):
    xe = e_ref[:, :HALF]
    xo = e_ref[:, HALF:]
    s = s_ref[...]
    c = c_ref[...]
    o_ref[:, :HALF] = xe * c - xo * s
    o_ref[:, HALF:] = xe * s + xo * c


_rope = pl.pallas_call(
    _rope_body,
    grid=(B // RB,),
    in_specs=[
        pl.BlockSpec((RB, DIM), lambda i: (i, 0)),
        pl.BlockSpec((RB, HALF), lambda i: (i % (SEQ // RB), 0)),
        pl.BlockSpec((RB, HALF), lambda i: (i % (SEQ // RB), 0)),
    ],
    out_specs=pl.BlockSpec((RB, DIM), lambda i: (i, 0)),
    out_shape=jax.ShapeDtypeStruct((B, DIM), jnp.float32),
)


def kernel(x, W):
    idx = x.reshape(NW, NCH, CH)
    emb = _sc_gather(W, idx)
    out = _rope(emb, jnp.asarray(_SIN), jnp.asarray(_COS))
    return out.reshape(BATCH, SEQ, DIM)
